# baseline (speedup 1.0000x reference)
import functools

import jax
import jax.numpy as jnp
from jax import lax
from jax.experimental import pallas as pl
from jax.experimental.pallas import tpu as pltpu
from jax.experimental.pallas import tpu_sc as plsc

_Q = 1024
_D = 64
_KB = 2000
_HB = _KB // 2


def _argmin_body(q2_ref, qsq_ref, a2_ref, idx_out_ref, best_d_ref, best_i_ref):
    i = pl.program_id(0)
    q2 = q2_ref[...]
    q_sq = qsq_ref[...]
    a2 = a2_ref[...]
    col2 = lax.broadcasted_iota(jnp.int32, (_Q, _HB), 1) * 2

    d2s = []
    for h in range(2):
        a_h = a2[:, h * _D:(h + 1) * _D]
        k_sq = jnp.sum(a_h * a_h, axis=1)
        dot2 = lax.dot_general(q2, a_h, (((1,), (1,)), ((), ())),
                               preferred_element_type=jnp.float32)
        d2s.append((q_sq - dot2) + k_sq[None, :])

    m2 = jnp.minimum(jnp.min(d2s[0], axis=1, keepdims=True),
                     jnp.min(d2s[1], axis=1, keepdims=True))
    m2 = jnp.maximum(m2, 0.0)
    bmin = jnp.sqrt(m2)
    thr = m2
    x = m2
    for _ in range(3):
        x = lax.bitcast_convert_type(
            lax.bitcast_convert_type(x, jnp.int32) + 1, jnp.float32)
        thr = jnp.where(jnp.sqrt(x) == bmin, x, thr)
    bidx = jnp.minimum(
        jnp.min(jnp.where(d2s[0] <= thr, col2, _KB), axis=1, keepdims=True),
        jnp.min(jnp.where(d2s[1] <= thr, col2 + 1, _KB), axis=1, keepdims=True))
    bidx = bidx + i * _KB

    @pl.when(i == 0)
    def _():
        best_d_ref[...] = bmin
        best_i_ref[...] = bidx

    @pl.when(i > 0)
    def _():
        upd = bmin < best_d_ref[...]
        best_d_ref[...] = jnp.where(upd, bmin, best_d_ref[...])
        best_i_ref[...] = jnp.where(upd, bidx, best_i_ref[...])

    @pl.when(i == pl.num_programs(0) - 1)
    def _():
        idx_out_ref[...] = best_i_ref[...]


def _tc_argmin(q2, q_sq, as2):
    npair = as2.shape[0]
    return pl.pallas_call(
        _argmin_body,
        grid=(npair // _HB,),
        in_specs=[
            pl.BlockSpec((_Q, _D), lambda i: (0, 0)),
            pl.BlockSpec((_Q, 1), lambda i: (0, 0)),
            pl.BlockSpec((_HB, 2 * _D), lambda i: (i, 0)),
        ],
        out_specs=pl.BlockSpec((_Q, 1), lambda i: (0, 0)),
        out_shape=jax.ShapeDtypeStruct((_Q, 1), jnp.int32),
        scratch_shapes=[
            pltpu.VMEM((_Q, 1), jnp.float32),
            pltpu.VMEM((_Q, 1), jnp.int32),
        ],
    )(q2, q_sq, as2)


def _sc_gather(as2, pair_idx):
    info = plsc.get_sparse_core_info()
    nw = info.num_cores * info.num_subcores
    bpw = _Q // nw
    nc = info.num_cores
    mesh = plsc.VectorSubcoreMesh(core_axis_name="c", subcore_axis_name="s")

    @functools.partial(
        pl.kernel,
        mesh=mesh,
        out_type=jax.ShapeDtypeStruct((_Q, 2 * _D), jnp.float32),
        scratch_types=[
            pltpu.VMEM((bpw,), jnp.int32),
            pltpu.VMEM((bpw, 2 * _D), jnp.float32),
            pltpu.SemaphoreType.DMA,
        ],
    )
    def gather(table_hbm, idx_hbm, out_hbm, idx_v, rows_v, sem):
        wid = lax.axis_index("s") * nc + lax.axis_index("c")
        base = wid * bpw
        pltpu.sync_copy(idx_hbm.at[pl.ds(base, bpw)], idx_v)
        pltpu.async_copy(table_hbm.at[idx_v], rows_v, sem).wait()
        pltpu.sync_copy(rows_v, out_hbm.at[pl.ds(base, bpw)])

    return gather(as2, pair_idx)


def kernel(pred_action, action_set):
    as2 = action_set.reshape(action_set.shape[0] // 2, 2 * _D)
    q2 = pred_action + pred_action
    q_sq = jnp.sum(pred_action * pred_action, axis=1, keepdims=True)
    idx = _tc_argmin(q2, q_sq, as2).reshape(_Q)
    pair_rows = _sc_gather(as2, idx // 2)
    even = (idx % 2 == 0)[:, None]
    return jnp.where(even, pair_rows[:, :_D], pair_rows[:, _D:])


# speedup vs baseline: 1.1437x; 1.1437x over previous
import functools

import jax
import jax.numpy as jnp
from jax import lax
from jax.experimental import pallas as pl
from jax.experimental.pallas import tpu as pltpu
from jax.experimental.pallas import tpu_sc as plsc

_Q = 1024
_D = 64
_KB = 2000


def _argmin_body(q2_ref, qsq_ref, a_ref, idx_out_ref, best_d_ref, best_i_ref):
    i = pl.program_id(0)
    q2 = q2_ref[...]
    q_sq = qsq_ref[...]
    a = a_ref[...]
    k_sq = jnp.sum(a * a, axis=1)
    dot2 = lax.dot_general(q2, a, (((1,), (1,)), ((), ())),
                           preferred_element_type=jnp.float32)
    d2 = (q_sq - dot2) + k_sq[None, :]
    m2 = jnp.maximum(jnp.min(d2, axis=1, keepdims=True), 0.0)
    bmin = jnp.sqrt(m2)
    thr = m2
    x = m2
    for _ in range(3):
        x = lax.bitcast_convert_type(
            lax.bitcast_convert_type(x, jnp.int32) + 1, jnp.float32)
        thr = jnp.where(jnp.sqrt(x) == bmin, x, thr)
    colf = lax.broadcasted_iota(jnp.int32, d2.shape, 1).astype(jnp.float32)
    bidxf = jnp.min(jnp.where(d2 <= thr, colf, float(_KB)),
                    axis=1, keepdims=True)
    bidxf = bidxf + jnp.float32(i * _KB)

    @pl.when(i == 0)
    def _():
        best_d_ref[...] = bmin
        best_i_ref[...] = bidxf

    @pl.when(i > 0)
    def _():
        upd = bmin < best_d_ref[...]
        best_d_ref[...] = jnp.where(upd, bmin, best_d_ref[...])
        best_i_ref[...] = jnp.where(upd, bidxf, best_i_ref[...])

    @pl.when(i == pl.num_programs(0) - 1)
    def _():
        idx_out_ref[...] = best_i_ref[...].astype(jnp.int32)


def _tc_argmin(pred_action, action_set):
    k = action_set.shape[0]
    q2 = pred_action + pred_action
    q_sq = jnp.sum(pred_action * pred_action, axis=1, keepdims=True)
    return pl.pallas_call(
        _argmin_body,
        grid=(k // _KB,),
        in_specs=[
            pl.BlockSpec((_Q, _D), lambda i: (0, 0)),
            pl.BlockSpec((_Q, 1), lambda i: (0, 0)),
            pl.BlockSpec((_KB, _D), lambda i: (i, 0)),
        ],
        out_specs=pl.BlockSpec((_Q, 1), lambda i: (0, 0)),
        out_shape=jax.ShapeDtypeStruct((_Q, 1), jnp.int32),
        scratch_shapes=[
            pltpu.VMEM((_Q, 1), jnp.float32),
            pltpu.VMEM((_Q, 1), jnp.float32),
        ],
    )(q2, q_sq, action_set)


def _sc_gather(table128, idx):
    info = plsc.get_sparse_core_info()
    nw = info.num_cores * info.num_subcores
    bpw = _Q // nw
    nc = info.num_cores
    mesh = plsc.VectorSubcoreMesh(core_axis_name="c", subcore_axis_name="s")

    @functools.partial(
        pl.kernel,
        mesh=mesh,
        out_type=jax.ShapeDtypeStruct((_Q, 128), jnp.float32),
        scratch_types=[
            pltpu.VMEM((bpw,), jnp.int32),
            pltpu.VMEM((bpw, 128), jnp.float32),
            pltpu.SemaphoreType.DMA,
        ],
    )
    def gather(table_hbm, idx_hbm, out_hbm, idx_v, rows_v, sem):
        wid = lax.axis_index("s") * nc + lax.axis_index("c")
        base = wid * bpw
        pltpu.sync_copy(idx_hbm.at[pl.ds(base, bpw)], idx_v)
        pltpu.async_copy(table_hbm.at[idx_v], rows_v, sem).wait()
        pltpu.sync_copy(rows_v, out_hbm.at[pl.ds(base, bpw)])

    return gather(table128, idx)


def kernel(pred_action, action_set):
    idx = _tc_argmin(pred_action, action_set).reshape(_Q)
    table128 = jnp.pad(action_set, ((0, 0), (0, 128 - _D)))
    return _sc_gather(table128, idx)[:, :_D]


# speedup vs baseline: 1.1438x; 1.0001x over previous
import functools

import jax
import jax.numpy as jnp
from jax import lax
from jax.experimental import pallas as pl
from jax.experimental.pallas import tpu as pltpu
from jax.experimental.pallas import tpu_sc as plsc

_Q = 1024
_D = 64
_KB = 2000


def _argmin_body(q2_ref, qsq_ref, a_ref, colf_ref, idx_out_ref,
                 best_d_ref, best_i_ref):
    i = pl.program_id(0)
    q2 = q2_ref[...]
    q_sq = qsq_ref[...]
    a = a_ref[...]
    k_sq = jnp.sum(a * a, axis=1)
    dot2 = lax.dot_general(q2, a, (((1,), (1,)), ((), ())),
                           preferred_element_type=jnp.float32)
    d2 = (q_sq - dot2) + k_sq[None, :]
    m2 = jnp.maximum(jnp.min(d2, axis=1, keepdims=True), 0.0)
    bmin = jnp.sqrt(m2)
    thr = m2
    x = m2
    for _ in range(3):
        x = lax.bitcast_convert_type(
            lax.bitcast_convert_type(x, jnp.int32) + 1, jnp.float32)
        thr = jnp.where(jnp.sqrt(x) == bmin, x, thr)
    colf = colf_ref[...]
    bidxf = jnp.min(jnp.where(d2 <= thr, colf, float(_KB)),
                    axis=1, keepdims=True)
    bidxf = bidxf + jnp.float32(i * _KB)

    @pl.when(i == 0)
    def _():
        best_d_ref[...] = bmin
        best_i_ref[...] = bidxf

    @pl.when(i > 0)
    def _():
        upd = bmin < best_d_ref[...]
        best_d_ref[...] = jnp.where(upd, bmin, best_d_ref[...])
        best_i_ref[...] = jnp.where(upd, bidxf, best_i_ref[...])

    @pl.when(i == pl.num_programs(0) - 1)
    def _():
        idx_out_ref[...] = best_i_ref[...].astype(jnp.int32)


def _tc_argmin(pred_action, action_set):
    k = action_set.shape[0]
    q2 = pred_action + pred_action
    q_sq = jnp.sum(pred_action * pred_action, axis=1, keepdims=True)
    colf = jnp.arange(_KB, dtype=jnp.float32).reshape(1, _KB)
    return pl.pallas_call(
        _argmin_body,
        grid=(k // _KB,),
        in_specs=[
            pl.BlockSpec((_Q, _D), lambda i: (0, 0)),
            pl.BlockSpec((_Q, 1), lambda i: (0, 0)),
            pl.BlockSpec((_KB, _D), lambda i: (i, 0)),
            pl.BlockSpec((1, _KB), lambda i: (0, 0)),
        ],
        out_specs=pl.BlockSpec((_Q, 1), lambda i: (0, 0)),
        out_shape=jax.ShapeDtypeStruct((_Q, 1), jnp.int32),
        scratch_shapes=[
            pltpu.VMEM((_Q, 1), jnp.float32),
            pltpu.VMEM((_Q, 1), jnp.float32),
        ],
    )(q2, q_sq, action_set, colf)


def _sc_gather(table128, idx):
    info = plsc.get_sparse_core_info()
    nw = info.num_cores * info.num_subcores
    bpw = _Q // nw
    nc = info.num_cores
    mesh = plsc.VectorSubcoreMesh(core_axis_name="c", subcore_axis_name="s")

    @functools.partial(
        pl.kernel,
        mesh=mesh,
        out_type=jax.ShapeDtypeStruct((_Q, 128), jnp.float32),
        scratch_types=[
            pltpu.VMEM((bpw,), jnp.int32),
            pltpu.VMEM((bpw, 128), jnp.float32),
            pltpu.SemaphoreType.DMA,
        ],
    )
    def gather(table_hbm, idx_hbm, out_hbm, idx_v, rows_v, sem):
        wid = lax.axis_index("s") * nc + lax.axis_index("c")
        base = wid * bpw
        pltpu.sync_copy(idx_hbm.at[pl.ds(base, bpw)], idx_v)
        pltpu.async_copy(table_hbm.at[idx_v], rows_v, sem).wait()
        pltpu.sync_copy(rows_v, out_hbm.at[pl.ds(base, bpw)])

    return gather(table128, idx)


def kernel(pred_action, action_set):
    idx = _tc_argmin(pred_action, action_set).reshape(_Q)
    table128 = jnp.pad(action_set, ((0, 0), (0, 128 - _D)))
    return _sc_gather(table128, idx)[:, :_D]


# speedup vs baseline: 1.2613x; 1.1027x over previous
import functools

import jax
import jax.numpy as jnp
from jax import lax
from jax.experimental import pallas as pl
from jax.experimental.pallas import tpu as pltpu
from jax.experimental.pallas import tpu_sc as plsc

_Q = 1024
_D = 64
_KB = 4000


def _argmin_body(q2_ref, qsq_ref, a_ref, colf_ref, idx_out_ref,
                 best_d_ref, best_i_ref):
    i = pl.program_id(0)
    q2 = q2_ref[...]
    q_sq = qsq_ref[...]
    a = a_ref[...]
    k_sq = jnp.sum(a * a, axis=1)
    dot2 = lax.dot_general(q2, a, (((1,), (1,)), ((), ())),
                           preferred_element_type=jnp.float32)
    d2 = (q_sq - dot2) + k_sq[None, :]
    m2 = jnp.maximum(jnp.min(d2, axis=1, keepdims=True), 0.0)
    bmin = jnp.sqrt(m2)
    thr = m2
    x = m2
    for _ in range(3):
        x = lax.bitcast_convert_type(
            lax.bitcast_convert_type(x, jnp.int32) + 1, jnp.float32)
        thr = jnp.where(jnp.sqrt(x) == bmin, x, thr)
    colf = colf_ref[...]
    bidxf = jnp.min(jnp.where(d2 <= thr, colf, float(_KB)),
                    axis=1, keepdims=True)
    bidxf = bidxf + jnp.float32(i * _KB)

    @pl.when(i == 0)
    def _():
        best_d_ref[...] = bmin
        best_i_ref[...] = bidxf

    @pl.when(i > 0)
    def _():
        upd = bmin < best_d_ref[...]
        best_d_ref[...] = jnp.where(upd, bmin, best_d_ref[...])
        best_i_ref[...] = jnp.where(upd, bidxf, best_i_ref[...])

    @pl.when(i == pl.num_programs(0) - 1)
    def _():
        idx_out_ref[...] = best_i_ref[...].astype(jnp.int32)


def _tc_argmin(pred_action, action_set):
    k = action_set.shape[0]
    q2 = pred_action + pred_action
    q_sq = jnp.sum(pred_action * pred_action, axis=1, keepdims=True)
    colf = jnp.arange(_KB, dtype=jnp.float32).reshape(1, _KB)
    return pl.pallas_call(
        _argmin_body,
        grid=(k // _KB,),
        in_specs=[
            pl.BlockSpec((_Q, _D), lambda i: (0, 0)),
            pl.BlockSpec((_Q, 1), lambda i: (0, 0)),
            pl.BlockSpec((_KB, _D), lambda i: (i, 0)),
            pl.BlockSpec((1, _KB), lambda i: (0, 0)),
        ],
        out_specs=pl.BlockSpec((_Q, 1), lambda i: (0, 0)),
        out_shape=jax.ShapeDtypeStruct((_Q, 1), jnp.int32),
        scratch_shapes=[
            pltpu.VMEM((_Q, 1), jnp.float32),
            pltpu.VMEM((_Q, 1), jnp.float32),
        ],
        compiler_params=pltpu.CompilerParams(
            vmem_limit_bytes=100 * 1024 * 1024),
    )(q2, q_sq, action_set, colf)


def _sc_gather(table128, idx):
    info = plsc.get_sparse_core_info()
    nw = info.num_cores * info.num_subcores
    bpw = _Q // nw
    nc = info.num_cores
    mesh = plsc.VectorSubcoreMesh(core_axis_name="c", subcore_axis_name="s")

    @functools.partial(
        pl.kernel,
        mesh=mesh,
        out_type=jax.ShapeDtypeStruct((_Q, 128), jnp.float32),
        scratch_types=[
            pltpu.VMEM((bpw,), jnp.int32),
            pltpu.VMEM((bpw, 128), jnp.float32),
            pltpu.SemaphoreType.DMA,
        ],
    )
    def gather(table_hbm, idx_hbm, out_hbm, idx_v, rows_v, sem):
        wid = lax.axis_index("s") * nc + lax.axis_index("c")
        base = wid * bpw
        pltpu.sync_copy(idx_hbm.at[pl.ds(base, bpw)], idx_v)
        pltpu.async_copy(table_hbm.at[idx_v], rows_v, sem).wait()
        pltpu.sync_copy(rows_v, out_hbm.at[pl.ds(base, bpw)])

    return gather(table128, idx)


def kernel(pred_action, action_set):
    idx = _tc_argmin(pred_action, action_set).reshape(_Q)
    table128 = jnp.pad(action_set, ((0, 0), (0, 128 - _D)))
    return _sc_gather(table128, idx)[:, :_D]


# speedup vs baseline: 1.2881x; 1.0212x over previous
import functools

import jax
import jax.numpy as jnp
from jax import lax
from jax.experimental import pallas as pl
from jax.experimental.pallas import tpu as pltpu
from jax.experimental.pallas import tpu_sc as plsc

_Q = 1024
_D = 64
_KB = 5000


def _argmin_body(q2_ref, qsq_ref, a_ref, colf_ref, idx_out_ref,
                 best_d_ref, best_i_ref):
    i = pl.program_id(0)
    q2 = q2_ref[...]
    q_sq = qsq_ref[...]
    a = a_ref[...]
    k_sq = jnp.sum(a * a, axis=1)
    dot2 = lax.dot_general(q2, a, (((1,), (1,)), ((), ())),
                           preferred_element_type=jnp.float32)
    d2 = (q_sq - dot2) + k_sq[None, :]
    m2 = jnp.maximum(jnp.min(d2, axis=1, keepdims=True), 0.0)
    bmin = jnp.sqrt(m2)
    thr = m2
    x = m2
    for _ in range(3):
        x = lax.bitcast_convert_type(
            lax.bitcast_convert_type(x, jnp.int32) + 1, jnp.float32)
        thr = jnp.where(jnp.sqrt(x) == bmin, x, thr)
    colf = colf_ref[...]
    bidxf = jnp.min(jnp.where(d2 <= thr, colf, float(_KB)),
                    axis=1, keepdims=True)
    bidxf = bidxf + jnp.float32(i * _KB)

    @pl.when(i == 0)
    def _():
        best_d_ref[...] = bmin
        best_i_ref[...] = bidxf

    @pl.when(i > 0)
    def _():
        upd = bmin < best_d_ref[...]
        best_d_ref[...] = jnp.where(upd, bmin, best_d_ref[...])
        best_i_ref[...] = jnp.where(upd, bidxf, best_i_ref[...])

    @pl.when(i == pl.num_programs(0) - 1)
    def _():
        idx_out_ref[...] = best_i_ref[...].astype(jnp.int32)


def _tc_argmin(pred_action, action_set):
    k = action_set.shape[0]
    q2 = pred_action + pred_action
    q_sq = jnp.sum(pred_action * pred_action, axis=1, keepdims=True)
    colf = jnp.arange(_KB, dtype=jnp.float32).reshape(1, _KB)
    return pl.pallas_call(
        _argmin_body,
        grid=(k // _KB,),
        in_specs=[
            pl.BlockSpec((_Q, _D), lambda i: (0, 0)),
            pl.BlockSpec((_Q, 1), lambda i: (0, 0)),
            pl.BlockSpec((_KB, _D), lambda i: (i, 0)),
            pl.BlockSpec((1, _KB), lambda i: (0, 0)),
        ],
        out_specs=pl.BlockSpec((_Q, 1), lambda i: (0, 0)),
        out_shape=jax.ShapeDtypeStruct((_Q, 1), jnp.int32),
        scratch_shapes=[
            pltpu.VMEM((_Q, 1), jnp.float32),
            pltpu.VMEM((_Q, 1), jnp.float32),
        ],
        compiler_params=pltpu.CompilerParams(
            vmem_limit_bytes=100 * 1024 * 1024),
    )(q2, q_sq, action_set, colf)


def _sc_gather(table128, idx):
    info = plsc.get_sparse_core_info()
    nw = info.num_cores * info.num_subcores
    bpw = _Q // nw
    nc = info.num_cores
    mesh = plsc.VectorSubcoreMesh(core_axis_name="c", subcore_axis_name="s")

    @functools.partial(
        pl.kernel,
        mesh=mesh,
        out_type=jax.ShapeDtypeStruct((_Q, 128), jnp.float32),
        scratch_types=[
            pltpu.VMEM((bpw,), jnp.int32),
            pltpu.VMEM((bpw, 128), jnp.float32),
            pltpu.SemaphoreType.DMA,
        ],
    )
    def gather(table_hbm, idx_hbm, out_hbm, idx_v, rows_v, sem):
        wid = lax.axis_index("s") * nc + lax.axis_index("c")
        base = wid * bpw
        pltpu.sync_copy(idx_hbm.at[pl.ds(base, bpw)], idx_v)
        pltpu.async_copy(table_hbm.at[idx_v], rows_v, sem).wait()
        pltpu.sync_copy(rows_v, out_hbm.at[pl.ds(base, bpw)])

    return gather(table128, idx)


def kernel(pred_action, action_set):
    idx = _tc_argmin(pred_action, action_set).reshape(_Q)
    table128 = jnp.pad(action_set, ((0, 0), (0, 128 - _D)))
    return _sc_gather(table128, idx)[:, :_D]


# speedup vs baseline: 1.3915x; 1.0803x over previous
import functools

import jax
import jax.numpy as jnp
from jax import lax
from jax.experimental import pallas as pl
from jax.experimental.pallas import tpu as pltpu
from jax.experimental.pallas import tpu_sc as plsc

_Q = 1024
_D = 64
_KB = 5000


def _argmin_body(q2_ref, qsq_ref, a_ref, colf_ref, idx_out_ref, t128_ref,
                 best_d_ref, best_i_ref):
    i = pl.program_id(0)
    q2 = q2_ref[...]
    q_sq = qsq_ref[...]
    a = a_ref[...]
    t128_ref[:, 0:_D] = a
    t128_ref[:, _D:2 * _D] = a
    k_sq = jnp.sum(a * a, axis=1)
    dot2 = lax.dot_general(q2, a, (((1,), (1,)), ((), ())),
                           preferred_element_type=jnp.float32)
    d2 = (q_sq - dot2) + k_sq[None, :]
    m2 = jnp.maximum(jnp.min(d2, axis=1, keepdims=True), 0.0)
    bmin = jnp.sqrt(m2)
    thr = m2
    x = m2
    for _ in range(3):
        x = lax.bitcast_convert_type(
            lax.bitcast_convert_type(x, jnp.int32) + 1, jnp.float32)
        thr = jnp.where(jnp.sqrt(x) == bmin, x, thr)
    colf = colf_ref[...]
    bidxf = jnp.min(jnp.where(d2 <= thr, colf, float(_KB)),
                    axis=1, keepdims=True)
    bidxf = bidxf + jnp.float32(i * _KB)

    @pl.when(i == 0)
    def _():
        best_d_ref[...] = bmin
        best_i_ref[...] = bidxf

    @pl.when(i > 0)
    def _():
        upd = bmin < best_d_ref[...]
        best_d_ref[...] = jnp.where(upd, bmin, best_d_ref[...])
        best_i_ref[...] = jnp.where(upd, bidxf, best_i_ref[...])

    @pl.when(i == pl.num_programs(0) - 1)
    def _():
        idx_out_ref[...] = best_i_ref[...].astype(jnp.int32)


def _tc_argmin(pred_action, action_set):
    k = action_set.shape[0]
    q2 = pred_action + pred_action
    q_sq = jnp.sum(pred_action * pred_action, axis=1, keepdims=True)
    colf = jnp.arange(_KB, dtype=jnp.float32).reshape(1, _KB)
    return pl.pallas_call(
        _argmin_body,
        grid=(k // _KB,),
        in_specs=[
            pl.BlockSpec((_Q, _D), lambda i: (0, 0)),
            pl.BlockSpec((_Q, 1), lambda i: (0, 0)),
            pl.BlockSpec((_KB, _D), lambda i: (i, 0)),
            pl.BlockSpec((1, _KB), lambda i: (0, 0)),
        ],
        out_specs=[
            pl.BlockSpec((_Q, 1), lambda i: (0, 0)),
            pl.BlockSpec((_KB, 2 * _D), lambda i: (i, 0)),
        ],
        out_shape=[
            jax.ShapeDtypeStruct((_Q, 1), jnp.int32),
            jax.ShapeDtypeStruct((k, 2 * _D), jnp.float32),
        ],
        scratch_shapes=[
            pltpu.VMEM((_Q, 1), jnp.float32),
            pltpu.VMEM((_Q, 1), jnp.float32),
        ],
        compiler_params=pltpu.CompilerParams(
            vmem_limit_bytes=100 * 1024 * 1024),
    )(q2, q_sq, action_set, colf)


def _sc_gather(table128, idx):
    info = plsc.get_sparse_core_info()
    nw = info.num_cores * info.num_subcores
    bpw = _Q // nw
    nc = info.num_cores
    mesh = plsc.VectorSubcoreMesh(core_axis_name="c", subcore_axis_name="s")

    @functools.partial(
        pl.kernel,
        mesh=mesh,
        out_type=jax.ShapeDtypeStruct((_Q, 128), jnp.float32),
        scratch_types=[
            pltpu.VMEM((bpw,), jnp.int32),
            pltpu.VMEM((bpw, 128), jnp.float32),
            pltpu.SemaphoreType.DMA,
        ],
    )
    def gather(table_hbm, idx_hbm, out_hbm, idx_v, rows_v, sem):
        wid = lax.axis_index("s") * nc + lax.axis_index("c")
        base = wid * bpw
        pltpu.sync_copy(idx_hbm.at[pl.ds(base, bpw)], idx_v)
        pltpu.async_copy(table_hbm.at[idx_v], rows_v, sem).wait()
        pltpu.sync_copy(rows_v, out_hbm.at[pl.ds(base, bpw)])

    return gather(table128, idx)


def kernel(pred_action, action_set):
    idx2, table128 = _tc_argmin(pred_action, action_set)
    return _sc_gather(table128, idx2.reshape(_Q))[:, :_D]
